# speedup vs baseline: 1.4417x; 1.4417x over previous
import functools

import jax
import jax.numpy as jnp
from jax.experimental import pallas as pl

N = 10000
E = 320000
D = 128
H = 64
C = 40
K = 16
ALPHA = 0.05


def _mlp_body(x_ref, w1_ref, b1_ref, w2_ref, b2_ref, o_ref):
    h = jnp.maximum(
        jnp.dot(x_ref[...], w1_ref[...], preferred_element_type=jnp.float32)
        + b1_ref[...],
        0.0,
    )
    o_ref[...] = (
        jnp.dot(h, w2_ref[...], preferred_element_type=jnp.float32) + b2_ref[...]
    )


def _mlp(x, W1, b1, W2, b2):
    blk = 2000
    grid = (N // blk,)
    return pl.pallas_call(
        _mlp_body,
        grid=grid,
        in_specs=[
            pl.BlockSpec((blk, D), lambda i: (i, 0)),
            pl.BlockSpec((D, H), lambda i: (0, 0)),
            pl.BlockSpec((1, H), lambda i: (0, 0)),
            pl.BlockSpec((H, C), lambda i: (0, 0)),
            pl.BlockSpec((1, C), lambda i: (0, 0)),
        ],
        out_specs=pl.BlockSpec((blk, C), lambda i: (i, 0)),
        out_shape=jax.ShapeDtypeStruct((N, C), jnp.float32),
    )(x, W1, b1.reshape(1, H), W2, b2.reshape(1, C))


def kernel(x, edge_index, W1, b1, W2, b2):
    h = _mlp(x, W1, b1, W2, b2)
    row = edge_index[0]
    col = edge_index[1]
    deg = jax.ops.segment_sum(jnp.ones((E,), jnp.float32), row, num_segments=N) + 1.0
    dis = deg ** -0.5
    norm = dis[row] * dis[col]
    deg_inv = 1.0 / deg
    out = h
    for _ in range(K):
        msg = norm[:, None] * out[row]
        agg = jax.ops.segment_sum(msg, col, num_segments=N)
        out = agg + deg_inv[:, None] * out + out
    res = ALPHA * h + (1.0 - ALPHA) * out / K
    return jax.nn.log_softmax(res, axis=1)


# speedup vs baseline: 16.0185x; 11.1106x over previous
import functools

import jax
import jax.numpy as jnp
from jax import lax
from jax.experimental import pallas as pl
from jax.experimental.pallas import tpu as pltpu
from jax.experimental.pallas import tpu_sc as plsc

N = 10000
E = 320000
D = 128
H = 64
C = 40
K = 16
ALPHA = 0.05

NS = 16
F = 48
R = 640
NP = NS * R
CH = 128
NCHUNK = -(-E // (NS * CH))
EPT = NCHUNK * CH
EPAD = EPT * NS

_mesh = plsc.VectorSubcoreMesh(
    core_axis_name="c", subcore_axis_name="s", num_cores=1
)
_sc_params = pltpu.CompilerParams(use_tc_tiling_on_sc=False)


@functools.partial(
    pl.kernel,
    out_type=jax.ShapeDtypeStruct((NP,), jnp.float32),
    mesh=_mesh,
    scratch_types=[
        pltpu.VMEM_SHARED((NP,), jnp.float32),
        pltpu.VMEM((NCHUNK, CH), jnp.int32),
        pltpu.VMEM((CH,), jnp.float32),
        pltpu.VMEM((R,), jnp.float32),
    ],
    compiler_params=_sc_params,
)
def _deg_kernel(rows_hbm, deg_hbm, deg_sp, row_t, ones_v, zbuf):
    t = lax.axis_index("s")
    sl = pl.ds(t * R, R)
    pltpu.sync_copy(rows_hbm.at[t], row_t)

    def fill(i, _):
        zbuf[pl.ds(i * 16, 16)] = jnp.zeros((16,), jnp.float32)
        return 0

    lax.fori_loop(0, R // 16, fill, 0)

    def fill1(i, _):
        ones_v[pl.ds(i * 16, 16)] = jnp.ones((16,), jnp.float32)
        return 0

    lax.fori_loop(0, CH // 16, fill1, 0)
    pltpu.sync_copy(zbuf, deg_sp.at[sl])
    plsc.subcore_barrier()

    def chunk(c, _):
        pltpu.sync_copy(ones_v, deg_sp.at[row_t.at[c]], add=True)
        return 0

    lax.fori_loop(0, NCHUNK, chunk, 0)
    plsc.subcore_barrier()
    pltpu.sync_copy(deg_sp.at[sl], deg_hbm.at[sl])


@functools.partial(
    pl.kernel,
    out_type=jax.ShapeDtypeStruct((NP, F), jnp.float32),
    mesh=_mesh,
    scratch_types=[
        pltpu.VMEM_SHARED((NP, F), jnp.float32),
        pltpu.VMEM_SHARED((NP, F), jnp.float32),
        pltpu.VMEM((NCHUNK, CH), jnp.int32),
        pltpu.VMEM((NCHUNK, CH), jnp.int32),
        pltpu.VMEM((CH, F), jnp.float32),
        pltpu.VMEM((CH, F), jnp.float32),
        pltpu.VMEM((R,), jnp.float32),
        pltpu.SemaphoreType.DMA,
    ],
    compiler_params=_sc_params,
)
def _prop_kernel(
    v0_hbm, dinv_hbm, rows_hbm, cols_hbm, vout_hbm,
    v_sp, w_sp, row_t, col_t, gbuf, ubuf, dbuf, gsem,
):
    t = lax.axis_index("s")
    sl = pl.ds(t * R, R)
    z16 = jnp.zeros((16,), jnp.float32)
    NSUB = R // CH

    pltpu.sync_copy(rows_hbm.at[t], row_t)
    pltpu.sync_copy(cols_hbm.at[t], col_t)
    pltpu.sync_copy(dinv_hbm.at[sl], dbuf)
    pltpu.sync_copy(v0_hbm.at[sl], v_sp.at[sl])

    def zrow(r, _):
        for j in range(F // 16):
            ubuf[r, pl.ds(j * 16, 16)] = z16
        return 0

    lax.fori_loop(0, CH, zrow, 0)

    def zsub(s, _):
        pltpu.sync_copy(ubuf, w_sp.at[pl.ds(t * R + s * CH, CH)])
        return 0

    lax.fori_loop(0, NSUB, zsub, 0)
    plsc.subcore_barrier()

    def hop(k, _):
        def chunk(c, _):
            pltpu.async_copy(v_sp.at[row_t.at[c]], gbuf, gsem).wait()
            pltpu.sync_copy(gbuf, w_sp.at[col_t.at[c]], add=True)
            return 0

        lax.fori_loop(0, NCHUNK, chunk, 0)
        plsc.subcore_barrier()

        def sub(s, _):
            base = pl.ds(t * R + s * CH, CH)
            pltpu.sync_copy(v_sp.at[base], gbuf)
            pltpu.sync_copy(w_sp.at[base], ubuf)

            def upd(i, _):
                d16 = dbuf[pl.ds(s * CH + i * 16, 16)]
                for l in range(16):
                    r = i * 16 + l
                    d = d16[l]
                    sc = 1.0 + d
                    for j in range(F // 16):
                        ds_ = pl.ds(j * 16, 16)
                        gbuf[r, ds_] = sc * gbuf[r, ds_] + d * ubuf[r, ds_]
                        ubuf[r, ds_] = z16
                return 0

            lax.fori_loop(0, CH // 16, upd, 0)
            pltpu.sync_copy(gbuf, v_sp.at[base])
            pltpu.sync_copy(ubuf, w_sp.at[base])
            return 0

        lax.fori_loop(0, NSUB, sub, 0)
        plsc.subcore_barrier()
        return 0

    lax.fori_loop(0, K, hop, 0)
    pltpu.sync_copy(v_sp.at[sl], vout_hbm.at[sl])


def _head_body(x_ref, w1_ref, b1_ref, w2_ref, b2_ref, deg_ref, h_ref, v0_ref, di_ref):
    hmid = jnp.maximum(
        jnp.dot(x_ref[...], w1_ref[...], preferred_element_type=jnp.float32)
        + b1_ref[...],
        0.0,
    )
    h = jnp.dot(hmid, w2_ref[...], preferred_element_type=jnp.float32) + b2_ref[...]
    hp = jnp.concatenate([h, jnp.zeros((NP, F - C), jnp.float32)], axis=1)
    h_ref[...] = hp
    deg = deg_ref[...] + 1.0
    dis = lax.rsqrt(deg)
    di_ref[...] = 1.0 / deg
    rowid = lax.broadcasted_iota(jnp.int32, (NP, 1), 0)
    v0_ref[...] = jnp.where(rowid < N, dis * hp, 0.0)


def _head(xp, W1, b1, W2, b2, deg_raw):
    return pl.pallas_call(
        _head_body,
        out_shape=(
            jax.ShapeDtypeStruct((NP, F), jnp.float32),
            jax.ShapeDtypeStruct((NP, F), jnp.float32),
            jax.ShapeDtypeStruct((NP, 1), jnp.float32),
        ),
    )(xp, W1, b1.reshape(1, H), W2, b2.reshape(1, C), deg_raw.reshape(NP, 1))


def _tail_body(h_ref, v_ref, di_ref, o_ref):
    sq = lax.rsqrt(di_ref[pl.ds(0, N), :])
    h = h_ref[pl.ds(0, N), pl.ds(0, C)]
    v = v_ref[pl.ds(0, N), pl.ds(0, C)]
    res = ALPHA * h + ((1.0 - ALPHA) / K) * sq * v
    m = jnp.max(res, axis=1, keepdims=True)
    ex = jnp.exp(res - m)
    lse = jnp.log(jnp.sum(ex, axis=1, keepdims=True))
    o_ref[...] = res - m - lse


def _tail(h_pad, vK, deg_inv):
    return pl.pallas_call(
        _tail_body,
        out_shape=jax.ShapeDtypeStruct((N, C), jnp.float32),
    )(h_pad, vK, deg_inv)


def kernel(x, edge_index, W1, b1, W2, b2):
    pad_e = EPAD - E
    rows = jnp.concatenate(
        [edge_index[0], jnp.full((pad_e,), N, jnp.int32)]
    ).reshape(NS, NCHUNK, CH)
    cols = jnp.concatenate(
        [edge_index[1], jnp.full((pad_e,), N, jnp.int32)]
    ).reshape(NS, NCHUNK, CH)
    xp = jnp.pad(x, ((0, NP - N), (0, 0)))

    deg_raw = _deg_kernel(rows)
    h_pad, v0, deg_inv = _head(xp, W1, b1, W2, b2, deg_raw)
    vK = _prop_kernel(v0, deg_inv.reshape(NP), rows, cols)
    return _tail(h_pad, vK, deg_inv)


# speedup vs baseline: 23.5216x; 1.4684x over previous
import functools

import jax
import jax.numpy as jnp
from jax import lax
from jax.experimental import pallas as pl
from jax.experimental.pallas import tpu as pltpu
from jax.experimental.pallas import tpu_sc as plsc

N = 10000
E = 320000
D = 128
H = 64
C = 40
K = 16
ALPHA = 0.05

NS = 16
FA = 32
FB = 16
F = FA + FB
R = 640
NP = NS * R
CH = 128
NCHUNK = -(-E // (NS * CH))
EPT = NCHUNK * CH
EPAD = EPT * NS
NSUB = R // CH

_mesh1 = plsc.VectorSubcoreMesh(
    core_axis_name="c", subcore_axis_name="s", num_cores=1
)
_mesh2 = plsc.VectorSubcoreMesh(
    core_axis_name="c", subcore_axis_name="s", num_cores=2
)
_sc_params = pltpu.CompilerParams(use_tc_tiling_on_sc=False)


@functools.partial(
    pl.kernel,
    out_type=jax.ShapeDtypeStruct((NP,), jnp.float32),
    mesh=_mesh1,
    scratch_types=[
        pltpu.VMEM_SHARED((NP,), jnp.float32),
        pltpu.VMEM((NCHUNK, CH), jnp.int32),
        pltpu.VMEM((CH,), jnp.float32),
        pltpu.VMEM((R,), jnp.float32),
    ],
    compiler_params=_sc_params,
)
def _deg_kernel(rows_hbm, deg_hbm, deg_sp, row_t, ones_v, zbuf):
    t = lax.axis_index("s")
    sl = pl.ds(t * R, R)
    pltpu.sync_copy(rows_hbm.at[t], row_t)

    def fill(i, _):
        zbuf[pl.ds(i * 16, 16)] = jnp.zeros((16,), jnp.float32)
        return 0

    lax.fori_loop(0, R // 16, fill, 0)

    def fill1(i, _):
        ones_v[pl.ds(i * 16, 16)] = jnp.ones((16,), jnp.float32)
        return 0

    lax.fori_loop(0, CH // 16, fill1, 0)
    pltpu.sync_copy(zbuf, deg_sp.at[sl])
    plsc.subcore_barrier()

    def chunk(c, _):
        pltpu.sync_copy(ones_v, deg_sp.at[row_t.at[c]], add=True)
        return 0

    lax.fori_loop(0, NCHUNK, chunk, 0)
    plsc.subcore_barrier()
    pltpu.sync_copy(deg_sp.at[sl], deg_hbm.at[sl])


@functools.partial(
    pl.kernel,
    out_type=(
        jax.ShapeDtypeStruct((NP, FA), jnp.float32),
        jax.ShapeDtypeStruct((NP, FB), jnp.float32),
    ),
    mesh=_mesh2,
    scratch_types=[
        pltpu.VMEM_SHARED((NP, FA), jnp.float32),
        pltpu.VMEM_SHARED((NP, FA), jnp.float32),
        pltpu.VMEM_SHARED((NP, FB), jnp.float32),
        pltpu.VMEM_SHARED((NP, FB), jnp.float32),
        pltpu.VMEM((NCHUNK, CH), jnp.int32),
        pltpu.VMEM((NCHUNK, CH), jnp.int32),
        pltpu.VMEM((CH, FA), jnp.float32),
        pltpu.VMEM((CH, FA), jnp.float32),
        pltpu.VMEM((CH, FB), jnp.float32),
        pltpu.VMEM((CH, FB), jnp.float32),
        pltpu.VMEM((R,), jnp.float32),
        pltpu.SemaphoreType.DMA,
    ],
    compiler_params=_sc_params,
)
def _prop_kernel(
    v0a_hbm, v0b_hbm, dinv_hbm, rows_hbm, cols_hbm, va_hbm, vb_hbm,
    va_sp, wa_sp, vb_sp, wb_sp, row_t, col_t, ga, ua, gb, ub, dbuf, gsem,
):
    ci = lax.axis_index("c")
    t = lax.axis_index("s")
    sl = pl.ds(t * R, R)
    z16 = jnp.zeros((16,), jnp.float32)

    pltpu.sync_copy(rows_hbm.at[t], row_t)
    pltpu.sync_copy(cols_hbm.at[t], col_t)
    pltpu.sync_copy(dinv_hbm.at[sl], dbuf)

    def _init(fw, v0_hbm, v_sp, w_sp, ubuf):
        pltpu.sync_copy(v0_hbm.at[sl], v_sp.at[sl])

        def zrow(r, _):
            for j in range(fw // 16):
                ubuf[r, pl.ds(j * 16, 16)] = z16
            return 0

        lax.fori_loop(0, CH, zrow, 0)

        def zsub(s, _):
            pltpu.sync_copy(ubuf, w_sp.at[pl.ds(t * R + s * CH, CH)])
            return 0

        lax.fori_loop(0, NSUB, zsub, 0)

    @pl.when(ci == 0)
    def _():
        _init(FA, v0a_hbm, va_sp, wa_sp, ua)

    @pl.when(ci == 1)
    def _():
        _init(FB, v0b_hbm, vb_sp, wb_sp, ub)

    plsc.subcore_barrier()

    def _edge_chunk(c, v_sp, w_sp, gbuf):
        pltpu.async_copy(v_sp.at[row_t.at[c]], gbuf, gsem).wait()
        pltpu.sync_copy(gbuf, w_sp.at[col_t.at[c]], add=True)

    def _update_sub(fw, s, v_sp, w_sp, gbuf, ubuf):
        base = pl.ds(t * R + s * CH, CH)
        pltpu.sync_copy(v_sp.at[base], gbuf)
        pltpu.sync_copy(w_sp.at[base], ubuf)

        def upd(i, _):
            d16 = dbuf[pl.ds(s * CH + i * 16, 16)]
            for l in range(16):
                r = i * 16 + l
                d = d16[l]
                sc = 1.0 + d
                for j in range(fw // 16):
                    ds_ = pl.ds(j * 16, 16)
                    gbuf[r, ds_] = sc * gbuf[r, ds_] + d * ubuf[r, ds_]
                    ubuf[r, ds_] = z16
            return 0

        lax.fori_loop(0, CH // 16, upd, 0)
        pltpu.sync_copy(gbuf, v_sp.at[base])
        pltpu.sync_copy(ubuf, w_sp.at[base])

    def hop(k, _):
        def chunk(c, _):
            @pl.when(ci == 0)
            def _():
                _edge_chunk(c, va_sp, wa_sp, ga)

            @pl.when(ci == 1)
            def _():
                _edge_chunk(c, vb_sp, wb_sp, gb)

            return 0

        lax.fori_loop(0, NCHUNK, chunk, 0)
        plsc.subcore_barrier()

        def sub(s, _):
            @pl.when(ci == 0)
            def _():
                _update_sub(FA, s, va_sp, wa_sp, ga, ua)

            @pl.when(ci == 1)
            def _():
                _update_sub(FB, s, vb_sp, wb_sp, gb, ub)

            return 0

        lax.fori_loop(0, NSUB, sub, 0)
        plsc.subcore_barrier()
        return 0

    lax.fori_loop(0, K, hop, 0)

    @pl.when(ci == 0)
    def _():
        pltpu.sync_copy(va_sp.at[sl], va_hbm.at[sl])

    @pl.when(ci == 1)
    def _():
        pltpu.sync_copy(vb_sp.at[sl], vb_hbm.at[sl])


def _head_body(x_ref, w1_ref, b1_ref, w2_ref, b2_ref, deg_ref,
               h_ref, v0a_ref, v0b_ref, di_ref):
    hmid = jnp.maximum(
        jnp.dot(x_ref[...], w1_ref[...], preferred_element_type=jnp.float32)
        + b1_ref[...],
        0.0,
    )
    h = jnp.dot(hmid, w2_ref[...], preferred_element_type=jnp.float32) + b2_ref[...]
    hp = jnp.concatenate([h, jnp.zeros((NP, F - C), jnp.float32)], axis=1)
    h_ref[...] = hp
    deg = deg_ref[...] + 1.0
    dis = lax.rsqrt(deg)
    di_ref[...] = 1.0 / deg
    rowid = lax.broadcasted_iota(jnp.int32, (NP, 1), 0)
    v0 = jnp.where(rowid < N, dis * hp, 0.0)
    v0a_ref[...] = v0[:, :FA]
    v0b_ref[...] = v0[:, FA:]


def _head(xp, W1, b1, W2, b2, deg_raw):
    return pl.pallas_call(
        _head_body,
        out_shape=(
            jax.ShapeDtypeStruct((NP, F), jnp.float32),
            jax.ShapeDtypeStruct((NP, FA), jnp.float32),
            jax.ShapeDtypeStruct((NP, FB), jnp.float32),
            jax.ShapeDtypeStruct((NP, 1), jnp.float32),
        ),
    )(xp, W1, b1.reshape(1, H), W2, b2.reshape(1, C), deg_raw.reshape(NP, 1))


def _tail_body(h_ref, va_ref, vb_ref, di_ref, o_ref):
    sq = lax.rsqrt(di_ref[pl.ds(0, N), :])
    h = h_ref[pl.ds(0, N), pl.ds(0, C)]
    v = jnp.concatenate(
        [va_ref[pl.ds(0, N), :], vb_ref[pl.ds(0, N), pl.ds(0, C - FA)]], axis=1
    )
    res = ALPHA * h + ((1.0 - ALPHA) / K) * sq * v
    m = jnp.max(res, axis=1, keepdims=True)
    ex = jnp.exp(res - m)
    lse = jnp.log(jnp.sum(ex, axis=1, keepdims=True))
    o_ref[...] = res - m - lse


def _tail(h_pad, vKa, vKb, deg_inv):
    return pl.pallas_call(
        _tail_body,
        out_shape=jax.ShapeDtypeStruct((N, C), jnp.float32),
    )(h_pad, vKa, vKb, deg_inv)


def kernel(x, edge_index, W1, b1, W2, b2):
    pad_e = EPAD - E
    rows = jnp.concatenate(
        [edge_index[0], jnp.full((pad_e,), N, jnp.int32)]
    ).reshape(NS, NCHUNK, CH)
    cols = jnp.concatenate(
        [edge_index[1], jnp.full((pad_e,), N, jnp.int32)]
    ).reshape(NS, NCHUNK, CH)
    xp = jnp.pad(x, ((0, NP - N), (0, 0)))

    deg_raw = _deg_kernel(rows)
    h_pad, v0a, v0b, deg_inv = _head(xp, W1, b1, W2, b2, deg_raw)
    vKa, vKb = _prop_kernel(v0a, v0b, deg_inv.reshape(NP), rows, cols)
    return _tail(h_pad, vKa, vKb, deg_inv)


# speedup vs baseline: 29.4642x; 1.2526x over previous
import functools

import jax
import jax.numpy as jnp
from jax import lax
from jax.experimental import pallas as pl
from jax.experimental.pallas import tpu as pltpu
from jax.experimental.pallas import tpu_sc as plsc

N = 10000
E = 320000
D = 128
H = 64
C = 40
K = 16
ALPHA = 0.05

NS = 16
FA = 24
FB = 16
F = FA + FB
R = 640
NP = NS * R
CH = 128
NCHUNK = -(-E // (NS * CH))
EPT = NCHUNK * CH
EPAD = EPT * NS
NSUB = R // CH

_mesh1 = plsc.VectorSubcoreMesh(
    core_axis_name="c", subcore_axis_name="s", num_cores=1
)
_mesh2 = plsc.VectorSubcoreMesh(
    core_axis_name="c", subcore_axis_name="s", num_cores=2
)
_sc_params = pltpu.CompilerParams(use_tc_tiling_on_sc=False)


@functools.partial(
    pl.kernel,
    out_type=jax.ShapeDtypeStruct((NP,), jnp.float32),
    mesh=_mesh1,
    scratch_types=[
        pltpu.VMEM_SHARED((NP,), jnp.float32),
        pltpu.VMEM((NCHUNK, CH), jnp.int32),
        pltpu.VMEM((CH,), jnp.float32),
        pltpu.VMEM((R,), jnp.float32),
    ],
    compiler_params=_sc_params,
)
def _deg_kernel(rows_hbm, deg_hbm, deg_sp, row_t, ones_v, zbuf):
    t = lax.axis_index("s")
    sl = pl.ds(t * R, R)
    pltpu.sync_copy(rows_hbm.at[t], row_t)

    def fill(i, _):
        zbuf[pl.ds(i * 16, 16)] = jnp.zeros((16,), jnp.float32)
        return 0

    lax.fori_loop(0, R // 16, fill, 0)

    def fill1(i, _):
        ones_v[pl.ds(i * 16, 16)] = jnp.ones((16,), jnp.float32)
        return 0

    lax.fori_loop(0, CH // 16, fill1, 0)
    pltpu.sync_copy(zbuf, deg_sp.at[sl])
    plsc.subcore_barrier()

    def chunk(c, _):
        pltpu.sync_copy(ones_v, deg_sp.at[row_t.at[c]], add=True)
        return 0

    lax.fori_loop(0, NCHUNK, chunk, 0)
    plsc.subcore_barrier()
    pltpu.sync_copy(deg_sp.at[sl], deg_hbm.at[sl])


@functools.partial(
    pl.kernel,
    out_type=(
        jax.ShapeDtypeStruct((NP, FA), jnp.float32),
        jax.ShapeDtypeStruct((NP, FB), jnp.float32),
    ),
    mesh=_mesh2,
    scratch_types=[
        pltpu.VMEM_SHARED((NP, FA), jnp.float32),
        pltpu.VMEM_SHARED((NP, FA), jnp.float32),
        pltpu.VMEM_SHARED((NP, FB), jnp.float32),
        pltpu.VMEM_SHARED((NP, FB), jnp.float32),
        pltpu.VMEM((NCHUNK, CH), jnp.int32),
        pltpu.VMEM((NCHUNK, CH), jnp.int32),
        pltpu.VMEM((CH, FA), jnp.float32),
        pltpu.VMEM((CH, FA), jnp.float32),
        pltpu.VMEM((CH, FB), jnp.float32),
        pltpu.VMEM((CH, FB), jnp.float32),
        pltpu.VMEM((R,), jnp.float32),
        pltpu.SemaphoreType.DMA,
    ],
    compiler_params=_sc_params,
)
def _prop_kernel(
    v0a_hbm, v0b_hbm, dinv_hbm, rows_hbm, cols_hbm, va_hbm, vb_hbm,
    va_sp, wa_sp, vb_sp, wb_sp, row_t, col_t, ga, ua, gb, ub, dbuf, gsem,
):
    ci = lax.axis_index("c")
    t = lax.axis_index("s")
    sl = pl.ds(t * R, R)
    z16 = jnp.zeros((16,), jnp.float32)

    pltpu.sync_copy(rows_hbm.at[t], row_t)
    pltpu.sync_copy(cols_hbm.at[t], col_t)
    pltpu.sync_copy(dinv_hbm.at[sl], dbuf)

    def _vslices(fw):
        out = [(j * 16, 16) for j in range(fw // 16)]
        if fw % 16:
            out.append((fw - fw % 16, fw % 16))
        return out

    def _init(fw, v0_hbm, v_sp, w_sp, ubuf):
        pltpu.sync_copy(v0_hbm.at[sl], v_sp.at[sl])

        def zrow(r, _):
            for (o, n) in _vslices(fw):
                ubuf[r, pl.ds(o, n)] = jnp.zeros((n,), jnp.float32)
            return 0

        lax.fori_loop(0, CH, zrow, 0)

        def zsub(s, _):
            pltpu.sync_copy(ubuf, w_sp.at[pl.ds(t * R + s * CH, CH)])
            return 0

        lax.fori_loop(0, NSUB, zsub, 0)

    @pl.when(ci == 0)
    def _():
        _init(FA, v0a_hbm, va_sp, wa_sp, ua)

    @pl.when(ci == 1)
    def _():
        _init(FB, v0b_hbm, vb_sp, wb_sp, ub)

    plsc.subcore_barrier()

    def _edge_chunk(c, v_sp, w_sp, gbuf):
        pltpu.async_copy(v_sp.at[row_t.at[c]], gbuf, gsem).wait()
        pltpu.sync_copy(gbuf, w_sp.at[col_t.at[c]], add=True)

    def _update_sub(fw, s, v_sp, w_sp, gbuf, ubuf):
        base = pl.ds(t * R + s * CH, CH)
        pltpu.sync_copy(v_sp.at[base], gbuf)
        pltpu.sync_copy(w_sp.at[base], ubuf)

        def upd(i, _):
            d16 = dbuf[pl.ds(s * CH + i * 16, 16)]
            for l in range(16):
                r = i * 16 + l
                d = d16[l]
                sc = 1.0 + d
                for (o, n) in _vslices(fw):
                    ds_ = pl.ds(o, n)
                    gbuf[r, ds_] = sc * gbuf[r, ds_] + d * ubuf[r, ds_]
                    ubuf[r, ds_] = jnp.zeros((n,), jnp.float32)
            return 0

        lax.fori_loop(0, CH // 16, upd, 0)
        pltpu.sync_copy(gbuf, v_sp.at[base])
        pltpu.sync_copy(ubuf, w_sp.at[base])

    def hop(k, _):
        def chunk(c, _):
            @pl.when(ci == 0)
            def _():
                _edge_chunk(c, va_sp, wa_sp, ga)

            @pl.when(ci == 1)
            def _():
                _edge_chunk(c, vb_sp, wb_sp, gb)

            return 0

        lax.fori_loop(0, NCHUNK, chunk, 0)
        plsc.subcore_barrier()

        def sub(s, _):
            @pl.when(ci == 0)
            def _():
                _update_sub(FA, s, va_sp, wa_sp, ga, ua)

            @pl.when(ci == 1)
            def _():
                _update_sub(FB, s, vb_sp, wb_sp, gb, ub)

            return 0

        lax.fori_loop(0, NSUB, sub, 0)
        plsc.subcore_barrier()
        return 0

    lax.fori_loop(0, K, hop, 0)

    @pl.when(ci == 0)
    def _():
        pltpu.sync_copy(va_sp.at[sl], va_hbm.at[sl])

    @pl.when(ci == 1)
    def _():
        pltpu.sync_copy(vb_sp.at[sl], vb_hbm.at[sl])


def _head_body(x_ref, w1_ref, b1_ref, w2_ref, b2_ref, deg_ref,
               h_ref, v0a_ref, v0b_ref, di_ref):
    hmid = jnp.maximum(
        jnp.dot(x_ref[...], w1_ref[...], preferred_element_type=jnp.float32)
        + b1_ref[...],
        0.0,
    )
    h = jnp.dot(hmid, w2_ref[...], preferred_element_type=jnp.float32) + b2_ref[...]
    if F > C:
        hp = jnp.concatenate([h, jnp.zeros((NP, F - C), jnp.float32)], axis=1)
    else:
        hp = h
    h_ref[...] = hp
    deg = deg_ref[...] + 1.0
    dis = lax.rsqrt(deg)
    di_ref[...] = 1.0 / deg
    rowid = lax.broadcasted_iota(jnp.int32, (NP, 1), 0)
    v0 = jnp.where(rowid < N, dis * hp, 0.0)
    v0a_ref[...] = v0[:, :FA]
    v0b_ref[...] = v0[:, FA:]


def _head(xp, W1, b1, W2, b2, deg_raw):
    return pl.pallas_call(
        _head_body,
        out_shape=(
            jax.ShapeDtypeStruct((NP, F), jnp.float32),
            jax.ShapeDtypeStruct((NP, FA), jnp.float32),
            jax.ShapeDtypeStruct((NP, FB), jnp.float32),
            jax.ShapeDtypeStruct((NP, 1), jnp.float32),
        ),
    )(xp, W1, b1.reshape(1, H), W2, b2.reshape(1, C), deg_raw.reshape(NP, 1))


def _tail_body(h_ref, va_ref, vb_ref, di_ref, o_ref):
    sq = lax.rsqrt(di_ref[pl.ds(0, N), :])
    h = h_ref[pl.ds(0, N), pl.ds(0, C)]
    v = jnp.concatenate(
        [va_ref[pl.ds(0, N), :], vb_ref[pl.ds(0, N), pl.ds(0, C - FA)]], axis=1
    )
    res = ALPHA * h + ((1.0 - ALPHA) / K) * sq * v
    m = jnp.max(res, axis=1, keepdims=True)
    ex = jnp.exp(res - m)
    lse = jnp.log(jnp.sum(ex, axis=1, keepdims=True))
    o_ref[...] = res - m - lse


def _tail(h_pad, vKa, vKb, deg_inv):
    return pl.pallas_call(
        _tail_body,
        out_shape=jax.ShapeDtypeStruct((N, C), jnp.float32),
    )(h_pad, vKa, vKb, deg_inv)


def kernel(x, edge_index, W1, b1, W2, b2):
    pad_e = EPAD - E
    rows = jnp.concatenate(
        [edge_index[0], jnp.full((pad_e,), N, jnp.int32)]
    ).reshape(NS, NCHUNK, CH)
    cols = jnp.concatenate(
        [edge_index[1], jnp.full((pad_e,), N, jnp.int32)]
    ).reshape(NS, NCHUNK, CH)
    xp = jnp.pad(x, ((0, NP - N), (0, 0)))

    deg_raw = _deg_kernel(rows)
    h_pad, v0a, v0b, deg_inv = _head(xp, W1, b1, W2, b2, deg_raw)
    vKa, vKb = _prop_kernel(v0a, v0b, deg_inv.reshape(NP), rows, cols)
    return _tail(h_pad, vKa, vKb, deg_inv)


# speedup vs baseline: 41.6690x; 1.4142x over previous
import functools

import jax
import jax.numpy as jnp
from jax import lax
from jax.experimental import pallas as pl
from jax.experimental.pallas import tpu as pltpu
from jax.experimental.pallas import tpu_sc as plsc

N = 10000
E = 320000
D = 128
H = 64
C = 40
K = 16
ALPHA = 0.05

NS = 16
FA = 24
FB = 16
F = FA + FB
R = 640
NP = NS * R
CH = 128
NCHUNK = -(-E // (NS * CH))
EPT = NCHUNK * CH
EPAD = EPT * NS
NSUB = R // CH

_mesh1 = plsc.VectorSubcoreMesh(
    core_axis_name="c", subcore_axis_name="s", num_cores=1
)
_mesh2 = plsc.VectorSubcoreMesh(
    core_axis_name="c", subcore_axis_name="s", num_cores=2
)
_sc_params = pltpu.CompilerParams(use_tc_tiling_on_sc=False)


@functools.partial(
    pl.kernel,
    out_type=jax.ShapeDtypeStruct((NP,), jnp.float32),
    mesh=_mesh1,
    scratch_types=[
        pltpu.VMEM_SHARED((NP,), jnp.float32),
        pltpu.VMEM((NCHUNK, CH), jnp.int32),
        pltpu.VMEM((CH,), jnp.float32),
        pltpu.VMEM((R,), jnp.float32),
    ],
    compiler_params=_sc_params,
)
def _deg_kernel(rows_hbm, deg_hbm, deg_sp, row_t, ones_v, zbuf):
    t = lax.axis_index("s")
    sl = pl.ds(t * R, R)
    pltpu.sync_copy(rows_hbm.at[t], row_t)

    def fill(i, _):
        zbuf[pl.ds(i * 16, 16)] = jnp.zeros((16,), jnp.float32)
        return 0

    lax.fori_loop(0, R // 16, fill, 0)

    def fill1(i, _):
        ones_v[pl.ds(i * 16, 16)] = jnp.ones((16,), jnp.float32)
        return 0

    lax.fori_loop(0, CH // 16, fill1, 0)
    pltpu.sync_copy(zbuf, deg_sp.at[sl])
    plsc.subcore_barrier()

    def chunk(c, _):
        pltpu.sync_copy(ones_v, deg_sp.at[row_t.at[c]], add=True)
        return 0

    lax.fori_loop(0, NCHUNK, chunk, 0)
    plsc.subcore_barrier()
    pltpu.sync_copy(deg_sp.at[sl], deg_hbm.at[sl])


@functools.partial(
    pl.kernel,
    out_type=(
        jax.ShapeDtypeStruct((NP, FA), jnp.float32),
        jax.ShapeDtypeStruct((NP, FB), jnp.float32),
    ),
    mesh=_mesh2,
    scratch_types=[
        pltpu.VMEM_SHARED((NP, FA), jnp.float32),
        pltpu.VMEM_SHARED((NP, FA), jnp.float32),
        pltpu.VMEM_SHARED((NP, FB), jnp.float32),
        pltpu.VMEM_SHARED((NP, FB), jnp.float32),
        pltpu.VMEM((NCHUNK, CH), jnp.int32),
        pltpu.VMEM((NCHUNK, CH), jnp.int32),
        pltpu.VMEM((CH, FA), jnp.float32),
        pltpu.VMEM((CH, FA), jnp.float32),
        pltpu.VMEM((CH, FB), jnp.float32),
        pltpu.VMEM((CH, FB), jnp.float32),
        pltpu.VMEM((R,), jnp.float32),
        pltpu.SemaphoreType.DMA,
    ],
    compiler_params=_sc_params,
)
def _prop_kernel(
    v0a_hbm, v0b_hbm, dinv_hbm, rows_hbm, cols_hbm, va_hbm, vb_hbm,
    va_sp, wa_sp, vb_sp, wb_sp, row_t, col_t, ga, ua, gb, ub, dbuf, gsem,
):
    ci = lax.axis_index("c")
    t = lax.axis_index("s")
    sl = pl.ds(t * R, R)
    z16 = jnp.zeros((16,), jnp.float32)

    pltpu.sync_copy(rows_hbm.at[t], row_t)
    pltpu.sync_copy(cols_hbm.at[t], col_t)
    pltpu.sync_copy(dinv_hbm.at[sl], dbuf)

    def _vslices(fw):
        out = [(j * 16, 16) for j in range(fw // 16)]
        if fw % 16:
            out.append((fw - fw % 16, fw % 16))
        return out

    def _init(fw, v0_hbm, v_sp, w_sp, ubuf):
        pltpu.sync_copy(v0_hbm.at[sl], v_sp.at[sl])

        def zrow(r, _):
            for (o, n) in _vslices(fw):
                ubuf[r, pl.ds(o, n)] = jnp.zeros((n,), jnp.float32)
            return 0

        lax.fori_loop(0, CH, zrow, 0)

        def zsub(s, _):
            pltpu.sync_copy(ubuf, w_sp.at[pl.ds(t * R + s * CH, CH)])
            return 0

        lax.fori_loop(0, NSUB, zsub, 0)

    @pl.when(ci == 0)
    def _():
        _init(FA, v0a_hbm, va_sp, wa_sp, ua)

    @pl.when(ci == 1)
    def _():
        _init(FB, v0b_hbm, vb_sp, wb_sp, ub)

    plsc.subcore_barrier()

    def _run_chunks(v_sp, w_sp, g0, g1, dummy):
        pltpu.async_copy(v_sp.at[row_t.at[0]], g0, gsem)

        def pair(p, _):
            c0 = 2 * p
            pltpu.async_copy(v_sp.at[row_t.at[c0 + 1]], g1, gsem)
            pltpu.make_async_copy(dummy, g0, gsem).wait()
            pltpu.sync_copy(g0, w_sp.at[col_t.at[c0]], add=True)
            pltpu.async_copy(v_sp.at[row_t.at[c0 + 2]], g0, gsem)
            pltpu.make_async_copy(dummy, g1, gsem).wait()
            pltpu.sync_copy(g1, w_sp.at[col_t.at[c0 + 1]], add=True)
            return 0

        lax.fori_loop(0, NCHUNK // 2, pair, 0)
        pltpu.make_async_copy(dummy, g0, gsem).wait()
        pltpu.sync_copy(g0, w_sp.at[col_t.at[NCHUNK - 1]], add=True)

    def _update_sub(fw, s, v_sp, w_sp, gbuf, ubuf):
        base = pl.ds(t * R + s * CH, CH)
        pltpu.sync_copy(v_sp.at[base], gbuf)
        pltpu.sync_copy(w_sp.at[base], ubuf)

        def upd(i, _):
            d16 = dbuf[pl.ds(s * CH + i * 16, 16)]
            for l in range(16):
                r = i * 16 + l
                d = d16[l]
                sc = 1.0 + d
                for (o, n) in _vslices(fw):
                    ds_ = pl.ds(o, n)
                    gbuf[r, ds_] = sc * gbuf[r, ds_] + d * ubuf[r, ds_]
                    ubuf[r, ds_] = jnp.zeros((n,), jnp.float32)
            return 0

        lax.fori_loop(0, CH // 16, upd, 0)
        pltpu.sync_copy(gbuf, v_sp.at[base])
        pltpu.sync_copy(ubuf, w_sp.at[base])

    def hop(k, _):
        @pl.when(ci == 0)
        def _():
            _run_chunks(va_sp, wa_sp, ga, ua, v0a_hbm.at[pl.ds(0, CH)])

        @pl.when(ci == 1)
        def _():
            _run_chunks(vb_sp, wb_sp, gb, ub, v0b_hbm.at[pl.ds(0, CH)])

        plsc.subcore_barrier()

        def sub(s, _):
            @pl.when(ci == 0)
            def _():
                _update_sub(FA, s, va_sp, wa_sp, ga, ua)

            @pl.when(ci == 1)
            def _():
                _update_sub(FB, s, vb_sp, wb_sp, gb, ub)

            return 0

        lax.fori_loop(0, NSUB, sub, 0)
        plsc.subcore_barrier()
        return 0

    lax.fori_loop(0, K, hop, 0)

    @pl.when(ci == 0)
    def _():
        pltpu.sync_copy(va_sp.at[sl], va_hbm.at[sl])

    @pl.when(ci == 1)
    def _():
        pltpu.sync_copy(vb_sp.at[sl], vb_hbm.at[sl])


def _head_body(x_ref, w1_ref, b1_ref, w2_ref, b2_ref, deg_ref,
               h_ref, v0a_ref, v0b_ref, di_ref):
    hmid = jnp.maximum(
        jnp.dot(x_ref[...], w1_ref[...], preferred_element_type=jnp.float32)
        + b1_ref[...],
        0.0,
    )
    h = jnp.dot(hmid, w2_ref[...], preferred_element_type=jnp.float32) + b2_ref[...]
    if F > C:
        hp = jnp.concatenate([h, jnp.zeros((NP, F - C), jnp.float32)], axis=1)
    else:
        hp = h
    h_ref[...] = hp
    deg = deg_ref[...] + 1.0
    dis = lax.rsqrt(deg)
    di_ref[...] = 1.0 / deg
    rowid = lax.broadcasted_iota(jnp.int32, (NP, 1), 0)
    v0 = jnp.where(rowid < N, dis * hp, 0.0)
    v0a_ref[...] = v0[:, :FA]
    v0b_ref[...] = v0[:, FA:]


def _head(xp, W1, b1, W2, b2, deg_raw):
    return pl.pallas_call(
        _head_body,
        out_shape=(
            jax.ShapeDtypeStruct((NP, F), jnp.float32),
            jax.ShapeDtypeStruct((NP, FA), jnp.float32),
            jax.ShapeDtypeStruct((NP, FB), jnp.float32),
            jax.ShapeDtypeStruct((NP, 1), jnp.float32),
        ),
    )(xp, W1, b1.reshape(1, H), W2, b2.reshape(1, C), deg_raw.reshape(NP, 1))


def _tail_body(h_ref, va_ref, vb_ref, di_ref, o_ref):
    sq = lax.rsqrt(di_ref[pl.ds(0, N), :])
    h = h_ref[pl.ds(0, N), pl.ds(0, C)]
    v = jnp.concatenate(
        [va_ref[pl.ds(0, N), :], vb_ref[pl.ds(0, N), pl.ds(0, C - FA)]], axis=1
    )
    res = ALPHA * h + ((1.0 - ALPHA) / K) * sq * v
    m = jnp.max(res, axis=1, keepdims=True)
    ex = jnp.exp(res - m)
    lse = jnp.log(jnp.sum(ex, axis=1, keepdims=True))
    o_ref[...] = res - m - lse


def _tail(h_pad, vKa, vKb, deg_inv):
    return pl.pallas_call(
        _tail_body,
        out_shape=jax.ShapeDtypeStruct((N, C), jnp.float32),
    )(h_pad, vKa, vKb, deg_inv)


def kernel(x, edge_index, W1, b1, W2, b2):
    pad_e = EPAD - E
    rows = jnp.concatenate(
        [edge_index[0], jnp.full((pad_e,), N, jnp.int32)]
    ).reshape(NS, NCHUNK, CH)
    cols = jnp.concatenate(
        [edge_index[1], jnp.full((pad_e,), N, jnp.int32)]
    ).reshape(NS, NCHUNK, CH)
    xp = jnp.pad(x, ((0, NP - N), (0, 0)))

    deg_raw = _deg_kernel(rows)
    h_pad, v0a, v0b, deg_inv = _head(xp, W1, b1, W2, b2, deg_raw)
    vKa, vKb = _prop_kernel(v0a, v0b, deg_inv.reshape(NP), rows, cols)
    return _tail(h_pad, vKa, vKb, deg_inv)


# speedup vs baseline: 43.1117x; 1.0346x over previous
import functools

import jax
import jax.numpy as jnp
from jax import lax
from jax.experimental import pallas as pl
from jax.experimental.pallas import tpu as pltpu
from jax.experimental.pallas import tpu_sc as plsc

N = 10000
E = 320000
D = 128
H = 64
C = 40
K = 16
ALPHA = 0.05

NS = 16
FA = 24
FB = 16
F = FA + FB
R = 640
NP = NS * R
CH = 128
NCHUNK = -(-E // (NS * CH))
EPT = NCHUNK * CH
EPAD = EPT * NS
NSUB = R // CH

_mesh1 = plsc.VectorSubcoreMesh(
    core_axis_name="c", subcore_axis_name="s", num_cores=1
)
_mesh2 = plsc.VectorSubcoreMesh(
    core_axis_name="c", subcore_axis_name="s", num_cores=2
)
_sc_params = pltpu.CompilerParams(use_tc_tiling_on_sc=False)


@functools.partial(
    pl.kernel,
    out_type=jax.ShapeDtypeStruct((NP,), jnp.float32),
    mesh=_mesh1,
    scratch_types=[
        pltpu.VMEM_SHARED((NP,), jnp.float32),
        pltpu.VMEM((NCHUNK, CH), jnp.int32),
        pltpu.VMEM((CH,), jnp.float32),
        pltpu.VMEM((R,), jnp.float32),
    ],
    compiler_params=_sc_params,
)
def _deg_kernel(rows_hbm, deg_hbm, deg_sp, row_t, ones_v, zbuf):
    t = lax.axis_index("s")
    sl = pl.ds(t * R, R)
    pltpu.sync_copy(rows_hbm.at[t], row_t)

    def fill(i, _):
        zbuf[pl.ds(i * 16, 16)] = jnp.zeros((16,), jnp.float32)
        return 0

    lax.fori_loop(0, R // 16, fill, 0)

    def fill1(i, _):
        ones_v[pl.ds(i * 16, 16)] = jnp.ones((16,), jnp.float32)
        return 0

    lax.fori_loop(0, CH // 16, fill1, 0)
    pltpu.sync_copy(zbuf, deg_sp.at[sl])
    plsc.subcore_barrier()

    def chunk(c, _):
        pltpu.sync_copy(ones_v, deg_sp.at[row_t.at[c]], add=True)
        return 0

    lax.fori_loop(0, NCHUNK, chunk, 0)
    plsc.subcore_barrier()
    pltpu.sync_copy(deg_sp.at[sl], deg_hbm.at[sl])


@functools.partial(
    pl.kernel,
    out_type=(
        jax.ShapeDtypeStruct((NP, FA), jnp.float32),
        jax.ShapeDtypeStruct((NP, FB), jnp.float32),
    ),
    mesh=_mesh2,
    scratch_types=[
        pltpu.VMEM_SHARED((NP, FA), jnp.float32),
        pltpu.VMEM_SHARED((NP, FA), jnp.float32),
        pltpu.VMEM_SHARED((NP, FB), jnp.float32),
        pltpu.VMEM_SHARED((NP, FB), jnp.float32),
        pltpu.VMEM((NCHUNK, CH), jnp.int32),
        pltpu.VMEM((NCHUNK, CH), jnp.int32),
        pltpu.VMEM((CH, FA), jnp.float32),
        pltpu.VMEM((CH, FA), jnp.float32),
        pltpu.VMEM((CH, FA), jnp.float32),
        pltpu.VMEM((CH, FA), jnp.float32),
        pltpu.VMEM((CH, FB), jnp.float32),
        pltpu.VMEM((CH, FB), jnp.float32),
        pltpu.VMEM((CH, FB), jnp.float32),
        pltpu.VMEM((CH, FB), jnp.float32),
        pltpu.VMEM((R,), jnp.float32),
        pltpu.SemaphoreType.DMA,
        pltpu.SemaphoreType.DMA,
    ],
    compiler_params=_sc_params,
)
def _prop_kernel(
    v0a_hbm, v0b_hbm, dinv_hbm, rows_hbm, cols_hbm, va_hbm, vb_hbm,
    va_sp, wa_sp, vb_sp, wb_sp, row_t, col_t,
    ga, ua, ga2, ua2, gb, ub, gb2, ub2, dbuf, gsem, ssem,
):
    ci = lax.axis_index("c")
    t = lax.axis_index("s")
    sl = pl.ds(t * R, R)
    z16 = jnp.zeros((16,), jnp.float32)

    pltpu.sync_copy(rows_hbm.at[t], row_t)
    pltpu.sync_copy(cols_hbm.at[t], col_t)
    pltpu.sync_copy(dinv_hbm.at[sl], dbuf)

    def _vslices(fw):
        out = [(j * 16, 16) for j in range(fw // 16)]
        if fw % 16:
            out.append((fw - fw % 16, fw % 16))
        return out

    def _init(fw, v0_hbm, v_sp, w_sp, ubuf):
        pltpu.sync_copy(v0_hbm.at[sl], v_sp.at[sl])

        def zrow(r, _):
            for (o, n) in _vslices(fw):
                ubuf[r, pl.ds(o, n)] = jnp.zeros((n,), jnp.float32)
            return 0

        lax.fori_loop(0, CH, zrow, 0)

        def zsub(s, _):
            pltpu.sync_copy(ubuf, w_sp.at[pl.ds(t * R + s * CH, CH)])
            return 0

        lax.fori_loop(0, NSUB, zsub, 0)

    @pl.when(ci == 0)
    def _():
        _init(FA, v0a_hbm, va_sp, wa_sp, ua)

    @pl.when(ci == 1)
    def _():
        _init(FB, v0b_hbm, vb_sp, wb_sp, ub)

    plsc.subcore_barrier()

    def _run_chunks(v_sp, w_sp, g0, g1, dummy):
        pltpu.async_copy(v_sp.at[row_t.at[0]], g0, gsem)

        def pair(p, _):
            c0 = 2 * p
            pltpu.async_copy(v_sp.at[row_t.at[c0 + 1]], g1, gsem)
            pltpu.make_async_copy(dummy, g0, gsem).wait()
            pltpu.sync_copy(g0, w_sp.at[col_t.at[c0]], add=True)
            pltpu.async_copy(v_sp.at[row_t.at[c0 + 2]], g0, gsem)
            pltpu.make_async_copy(dummy, g1, gsem).wait()
            pltpu.sync_copy(g1, w_sp.at[col_t.at[c0 + 1]], add=True)
            return 0

        lax.fori_loop(0, NCHUNK // 2, pair, 0)
        pltpu.make_async_copy(dummy, g0, gsem).wait()
        pltpu.sync_copy(g0, w_sp.at[col_t.at[NCHUNK - 1]], add=True)

    def _upd_block(fw, s, gbuf, ubuf):
        def upd(i, _):
            d16 = dbuf[pl.ds(s * CH + i * 16, 16)]
            for l in range(16):
                r = i * 16 + l
                d = d16[l]
                sc = 1.0 + d
                for (o, n) in _vslices(fw):
                    ds_ = pl.ds(o, n)
                    gbuf[r, ds_] = sc * gbuf[r, ds_] + d * ubuf[r, ds_]
                    ubuf[r, ds_] = jnp.zeros((n,), jnp.float32)
            return 0

        lax.fori_loop(0, CH // 16, upd, 0)

    def _update_all(fw, v_sp, w_sp, bufs, dummy):
        def base(s):
            return pl.ds(t * R + s * CH, CH)

        def drain(sem, gbuf):
            pltpu.make_async_copy(dummy, gbuf, sem).wait()
            pltpu.make_async_copy(dummy, gbuf, sem).wait()

        pltpu.async_copy(v_sp.at[base(0)], bufs[0][0], gsem)
        pltpu.async_copy(w_sp.at[base(0)], bufs[0][1], gsem)
        for s in range(NSUB):
            g_, u_ = bufs[s % 2]
            if s >= 1:
                drain(ssem, g_)
            if s + 1 < NSUB:
                gn, un = bufs[(s + 1) % 2]
                pltpu.async_copy(v_sp.at[base(s + 1)], gn, gsem)
                pltpu.async_copy(w_sp.at[base(s + 1)], un, gsem)
            drain(gsem, g_)
            _upd_block(fw, s, g_, u_)
            pltpu.async_copy(g_, v_sp.at[base(s)], ssem)
            pltpu.async_copy(u_, w_sp.at[base(s)], ssem)
        drain(ssem, bufs[(NSUB - 1) % 2][0])

    def hop(k, _):
        @pl.when(ci == 0)
        def _():
            _run_chunks(va_sp, wa_sp, ga, ua, v0a_hbm.at[pl.ds(0, CH)])

        @pl.when(ci == 1)
        def _():
            _run_chunks(vb_sp, wb_sp, gb, ub, v0b_hbm.at[pl.ds(0, CH)])

        plsc.subcore_barrier()

        @pl.when(ci == 0)
        def _():
            _update_all(FA, va_sp, wa_sp, ((ga, ua), (ga2, ua2)),
                        v0a_hbm.at[pl.ds(0, CH)])

        @pl.when(ci == 1)
        def _():
            _update_all(FB, vb_sp, wb_sp, ((gb, ub), (gb2, ub2)),
                        v0b_hbm.at[pl.ds(0, CH)])

        plsc.subcore_barrier()
        return 0

    lax.fori_loop(0, K, hop, 0)

    @pl.when(ci == 0)
    def _():
        pltpu.sync_copy(va_sp.at[sl], va_hbm.at[sl])

    @pl.when(ci == 1)
    def _():
        pltpu.sync_copy(vb_sp.at[sl], vb_hbm.at[sl])


def _head_body(x_ref, w1_ref, b1_ref, w2_ref, b2_ref, deg_ref,
               h_ref, v0a_ref, v0b_ref, di_ref):
    hmid = jnp.maximum(
        jnp.dot(x_ref[...], w1_ref[...], preferred_element_type=jnp.float32)
        + b1_ref[...],
        0.0,
    )
    h = jnp.dot(hmid, w2_ref[...], preferred_element_type=jnp.float32) + b2_ref[...]
    if F > C:
        hp = jnp.concatenate([h, jnp.zeros((NP, F - C), jnp.float32)], axis=1)
    else:
        hp = h
    h_ref[...] = hp
    deg = deg_ref[...] + 1.0
    dis = lax.rsqrt(deg)
    di_ref[...] = 1.0 / deg
    rowid = lax.broadcasted_iota(jnp.int32, (NP, 1), 0)
    v0 = jnp.where(rowid < N, dis * hp, 0.0)
    v0a_ref[...] = v0[:, :FA]
    v0b_ref[...] = v0[:, FA:]


def _head(xp, W1, b1, W2, b2, deg_raw):
    return pl.pallas_call(
        _head_body,
        out_shape=(
            jax.ShapeDtypeStruct((NP, F), jnp.float32),
            jax.ShapeDtypeStruct((NP, FA), jnp.float32),
            jax.ShapeDtypeStruct((NP, FB), jnp.float32),
            jax.ShapeDtypeStruct((NP, 1), jnp.float32),
        ),
    )(xp, W1, b1.reshape(1, H), W2, b2.reshape(1, C), deg_raw.reshape(NP, 1))


def _tail_body(h_ref, va_ref, vb_ref, di_ref, o_ref):
    sq = lax.rsqrt(di_ref[pl.ds(0, N), :])
    h = h_ref[pl.ds(0, N), pl.ds(0, C)]
    v = jnp.concatenate(
        [va_ref[pl.ds(0, N), :], vb_ref[pl.ds(0, N), pl.ds(0, C - FA)]], axis=1
    )
    res = ALPHA * h + ((1.0 - ALPHA) / K) * sq * v
    m = jnp.max(res, axis=1, keepdims=True)
    ex = jnp.exp(res - m)
    lse = jnp.log(jnp.sum(ex, axis=1, keepdims=True))
    o_ref[...] = res - m - lse


def _tail(h_pad, vKa, vKb, deg_inv):
    return pl.pallas_call(
        _tail_body,
        out_shape=jax.ShapeDtypeStruct((N, C), jnp.float32),
    )(h_pad, vKa, vKb, deg_inv)


def kernel(x, edge_index, W1, b1, W2, b2):
    pad_e = EPAD - E
    rows = jnp.concatenate(
        [edge_index[0], jnp.full((pad_e,), N, jnp.int32)]
    ).reshape(NS, NCHUNK, CH)
    cols = jnp.concatenate(
        [edge_index[1], jnp.full((pad_e,), N, jnp.int32)]
    ).reshape(NS, NCHUNK, CH)
    xp = jnp.pad(x, ((0, NP - N), (0, 0)))

    deg_raw = _deg_kernel(rows)
    h_pad, v0a, v0b, deg_inv = _head(xp, W1, b1, W2, b2, deg_raw)
    vKa, vKb = _prop_kernel(v0a, v0b, deg_inv.reshape(NP), rows, cols)
    return _tail(h_pad, vKa, vKb, deg_inv)
